# Initial kernel scaffold; baseline (speedup 1.0000x reference)
import jax
import jax.numpy as jnp
from jax.experimental import pallas as pl


def kernel(x, edge_index, pool_W, pool_b, fc1_W, fc1_b, fc2_W, fc2_b, mu):
    raise NotImplementedError("write your pallas kernel here")


# speedup vs baseline: 5.6928x; 5.6928x over previous
import functools

import jax
import jax.numpy as jnp
from jax import lax
from jax.experimental import pallas as pl
from jax.experimental.pallas import tpu as pltpu
from jax.experimental.pallas import tpu_sc as plsc

_CHUNK = 128
_NCORES = 2
_NSUB = 16
_NW = _NCORES * _NSUB
_LANES = 16
_BLK = 1000


def _stage1_body(mu_ref, x_ref, wT_ref, b_ref, h_ref, x3_ref):
    acc = jnp.dot(x_ref[...], wT_ref[...],
                  preferred_element_type=jnp.float32,
                  precision=lax.Precision.HIGHEST)
    h = jnp.maximum(acc + b_ref[...], 0.0)
    h_ref[...] = h
    mu = mu_ref[...]
    safe = jnp.where(h > 0.0, h, 1.0)
    x3_ref[...] = jnp.where(h > 0.0, jnp.exp(mu * jnp.log(safe)), 0.0)


def _stage3_body(imu_ref, h_ref, p_ref, f1T_ref, f2T_ref, bb_ref, o_ref):
    p = p_ref[...]
    s = p[0] + p[1]
    imu = imu_ref[...]
    safe = jnp.where(s > 0.0, s, 1.0)
    x2 = jnp.where(s > 0.0, jnp.exp(imu * jnp.log(safe)), 0.0)
    o_ref[...] = (jnp.dot(h_ref[...], f1T_ref[...],
                          preferred_element_type=jnp.float32,
                          precision=lax.Precision.HIGHEST)
                  + jnp.dot(x2, f2T_ref[...],
                            preferred_element_type=jnp.float32,
                            precision=lax.Precision.HIGHEST)
                  + bb_ref[...])


def _make_sc_scatter(n_pad, d, e):
    n_chunks = e // _CHUNK
    steps = (n_chunks + _NW - 1) // _NW
    rows_per_sub = n_pad // _NSUB
    mesh = plsc.VectorSubcoreMesh(core_axis_name="c", subcore_axis_name="s")

    @functools.partial(
        pl.kernel,
        mesh=mesh,
        out_type=jax.ShapeDtypeStruct((_NCORES * n_pad, d), jnp.float32),
        scratch_types=[
            pltpu.VMEM((_CHUNK,), jnp.int32),
            pltpu.VMEM((_CHUNK,), jnp.int32),
            pltpu.VMEM((_CHUNK, d), jnp.float32),
            pltpu.VMEM_SHARED((n_pad, d), jnp.float32),
            pltpu.SemaphoreType.DMA,
        ],
    )
    def scatter_kernel(src_hbm, dst_hbm, x3_hbm, out_hbm,
                       src_v, dst_v, rows_v, acc_sh, sem):
        c = lax.axis_index("c")
        s = lax.axis_index("s")
        w = s * _NCORES + c

        zrow = jnp.zeros((_LANES,), jnp.float32)

        @pl.loop(0, _CHUNK)
        def _(i):
            @pl.loop(0, d, step=_LANES)
            def _(j):
                rows_v[i, pl.ds(j, _LANES)] = zrow

        @pl.loop(0, rows_per_sub, step=_CHUNK)
        def _(r):
            pltpu.sync_copy(rows_v, acc_sh.at[pl.ds(s * rows_per_sub + r, _CHUNK)])

        plsc.subcore_barrier()

        @pl.loop(0, steps)
        def _(k):
            j = k * _NW + w

            @pl.when(j < n_chunks)
            def _():
                base = pl.multiple_of(j * _CHUNK, _CHUNK)
                pltpu.sync_copy(src_hbm.at[pl.ds(base, _CHUNK)], src_v)
                pltpu.sync_copy(dst_hbm.at[pl.ds(base, _CHUNK)], dst_v)
                pltpu.async_copy(x3_hbm.at[src_v], rows_v, sem).wait()
                pltpu.sync_copy(rows_v, acc_sh.at[dst_v], add=True)

        plsc.subcore_barrier()

        @pl.loop(0, rows_per_sub, step=_CHUNK)
        def _(r):
            row = s * rows_per_sub + r
            pltpu.sync_copy(acc_sh.at[pl.ds(row, _CHUNK)],
                            out_hbm.at[pl.ds(c * n_pad + row, _CHUNK)])

    return scatter_kernel


def kernel(x, edge_index, pool_W, pool_b, fc1_W, fc1_b, fc2_W, fc2_b, mu):
    n, d_in = x.shape
    d_pool = pool_W.shape[0]
    d_out = fc1_W.shape[0]
    e = edge_index.shape[1]
    n_pad = ((n + _CHUNK - 1) // _CHUNK) * _CHUNK
    if n_pad % _NSUB != 0 or (n_pad // _NSUB) % _CHUNK != 0:
        n_pad = ((n + _NSUB * _CHUNK - 1) // (_NSUB * _CHUNK)) * (_NSUB * _CHUNK)
    grid = n // _BLK

    mu_f = jnp.asarray(mu, jnp.float32).reshape(1, 1)
    mu_row = jnp.broadcast_to(mu_f, (1, d_pool))
    imu_row = jnp.broadcast_to(1.0 / mu_f, (1, d_pool))

    h, x3 = pl.pallas_call(
        _stage1_body,
        grid=(grid,),
        in_specs=[
            pl.BlockSpec((1, d_pool), lambda i: (0, 0)),
            pl.BlockSpec((_BLK, d_in), lambda i: (i, 0)),
            pl.BlockSpec((d_in, d_pool), lambda i: (0, 0)),
            pl.BlockSpec((1, d_pool), lambda i: (0, 0)),
        ],
        out_specs=[
            pl.BlockSpec((_BLK, d_pool), lambda i: (i, 0)),
            pl.BlockSpec((_BLK, d_pool), lambda i: (i, 0)),
        ],
        out_shape=[
            jax.ShapeDtypeStruct((n, d_pool), jnp.float32),
            jax.ShapeDtypeStruct((n, d_pool), jnp.float32),
        ],
    )(mu_row, x, pool_W.T, pool_b.reshape(1, -1))

    dst = edge_index[0]
    src = edge_index[1]
    agg_flat = _make_sc_scatter(n_pad, d_pool, e)(src, dst, x3)
    agg3 = agg_flat.reshape(_NCORES, n_pad, d_pool)

    out = pl.pallas_call(
        _stage3_body,
        grid=(grid,),
        in_specs=[
            pl.BlockSpec((1, d_pool), lambda i: (0, 0)),
            pl.BlockSpec((_BLK, d_pool), lambda i: (i, 0)),
            pl.BlockSpec((_NCORES, _BLK, d_pool), lambda i: (0, i, 0)),
            pl.BlockSpec((d_pool, d_out), lambda i: (0, 0)),
            pl.BlockSpec((d_pool, d_out), lambda i: (0, 0)),
            pl.BlockSpec((1, d_out), lambda i: (0, 0)),
        ],
        out_specs=pl.BlockSpec((_BLK, d_out), lambda i: (i, 0)),
        out_shape=jax.ShapeDtypeStruct((n, d_out), jnp.float32),
    )(imu_row, h, agg3, fc1_W.T, fc2_W.T, (fc1_b + fc2_b).reshape(1, -1))

    return out
